# initial kernel scaffold (unmeasured)
import jax
import jax.numpy as jnp
from jax import lax
from jax.experimental import pallas as pl
from jax.experimental.pallas import tpu as pltpu


def kernel(
    x,
):
    def body(*refs):
        pass

    out_shape = jax.ShapeDtypeStruct(..., jnp.float32)
    return pl.pallas_call(body, out_shape=out_shape)(...)


# baseline (device time: 8997 ns/iter reference)
import jax
import jax.numpy as jnp
from jax import lax
from jax.experimental import pallas as pl
from jax.experimental.pallas import tpu as pltpu

N_DEV = 4


def kernel(x):
    m, n = x.shape

    def body(x_ref, out_ref, lhalo_ref, rhalo_ref, send_sems, recv_sems):
        my_i = lax.axis_index("i")
        has_left = my_i > 0
        has_right = my_i < N_DEV - 1
        left = jnp.maximum(my_i - 1, 0)
        right = jnp.minimum(my_i + 1, N_DEV - 1)

        down = pltpu.make_async_remote_copy(
            src_ref=x_ref.at[pl.ds(m - 1, 1), :],
            dst_ref=lhalo_ref,
            send_sem=send_sems.at[0],
            recv_sem=recv_sems.at[0],
            device_id=(right,),
            device_id_type=pl.DeviceIdType.MESH,
        )
        up = pltpu.make_async_remote_copy(
            src_ref=x_ref.at[pl.ds(0, 1), :],
            dst_ref=rhalo_ref,
            send_sem=send_sems.at[1],
            recv_sem=recv_sems.at[1],
            device_id=(left,),
            device_id_type=pl.DeviceIdType.MESH,
        )

        @pl.when(has_right)
        def _():
            down.start()

        @pl.when(has_left)
        def _():
            up.start()

        out_ref[pl.ds(1, m - 2), :] = (
            0.25 * x_ref[pl.ds(0, m - 2), :]
            + 0.5 * x_ref[pl.ds(1, m - 2), :]
            + 0.25 * x_ref[pl.ds(2, m - 2), :]
        )

        @pl.when(jnp.logical_not(has_left))
        def _():
            out_ref[pl.ds(0, 1), :] = x_ref[pl.ds(0, 1), :]

        @pl.when(has_left)
        def _():
            down.wait_recv()
            out_ref[pl.ds(0, 1), :] = (
                0.25 * lhalo_ref[:, :]
                + 0.5 * x_ref[pl.ds(0, 1), :]
                + 0.25 * x_ref[pl.ds(1, 1), :]
            )
            up.wait_send()

        @pl.when(jnp.logical_not(has_right))
        def _():
            out_ref[pl.ds(m - 1, 1), :] = x_ref[pl.ds(m - 1, 1), :]

        @pl.when(has_right)
        def _():
            up.wait_recv()
            out_ref[pl.ds(m - 1, 1), :] = (
                0.25 * x_ref[pl.ds(m - 2, 1), :]
                + 0.5 * x_ref[pl.ds(m - 1, 1), :]
                + 0.25 * rhalo_ref[:, :]
            )
            down.wait_send()

    return pl.pallas_call(
        body,
        out_shape=jax.ShapeDtypeStruct((m, n), x.dtype),
        in_specs=[pl.BlockSpec(memory_space=pltpu.VMEM)],
        out_specs=pl.BlockSpec(memory_space=pltpu.VMEM),
        scratch_shapes=[
            pltpu.VMEM((1, n), x.dtype),
            pltpu.VMEM((1, n), x.dtype),
            pltpu.SemaphoreType.DMA((2,)),
            pltpu.SemaphoreType.DMA((2,)),
        ],
    )(x)


# device time: 6469 ns/iter; 1.3908x vs baseline; 1.3908x over previous
import jax
import jax.numpy as jnp
from jax import lax
from jax.experimental import pallas as pl
from jax.experimental.pallas import tpu as pltpu

N_DEV = 4


def kernel(x):
    m, n = x.shape

    def body(x_ref, out_ref, lhalo_ref, rhalo_ref, send_sems, recv_sems):
        my_i = lax.axis_index("i")
        has_left = my_i > 0
        has_right = my_i < N_DEV - 1
        left = jnp.maximum(my_i - 1, 0)
        right = jnp.minimum(my_i + 1, N_DEV - 1)

        down = pltpu.make_async_remote_copy(
            src_ref=x_ref.at[pl.ds(m - 1, 1), :],
            dst_ref=lhalo_ref,
            send_sem=send_sems.at[0],
            recv_sem=recv_sems.at[0],
            device_id=(right,),
            device_id_type=pl.DeviceIdType.MESH,
        )
        up = pltpu.make_async_remote_copy(
            src_ref=x_ref.at[pl.ds(0, 1), :],
            dst_ref=rhalo_ref,
            send_sem=send_sems.at[1],
            recv_sem=recv_sems.at[1],
            device_id=(left,),
            device_id_type=pl.DeviceIdType.MESH,
        )

        barrier_sem = pltpu.get_barrier_semaphore()

        @pl.when(has_left)
        def _():
            pl.semaphore_signal(
                barrier_sem, inc=1, device_id=(left,),
                device_id_type=pl.DeviceIdType.MESH,
            )

        @pl.when(has_right)
        def _():
            pl.semaphore_signal(
                barrier_sem, inc=1, device_id=(right,),
                device_id_type=pl.DeviceIdType.MESH,
            )

        n_nbrs = has_left.astype(jnp.int32) + has_right.astype(jnp.int32)
        pl.semaphore_wait(barrier_sem, n_nbrs)

        @pl.when(has_right)
        def _():
            down.start()

        @pl.when(has_left)
        def _():
            up.start()

        out_ref[pl.ds(1, m - 2), :] = (
            0.25 * x_ref[pl.ds(0, m - 2), :]
            + 0.5 * x_ref[pl.ds(1, m - 2), :]
            + 0.25 * x_ref[pl.ds(2, m - 2), :]
        )

        @pl.when(jnp.logical_not(has_left))
        def _():
            out_ref[pl.ds(0, 1), :] = x_ref[pl.ds(0, 1), :]

        @pl.when(has_left)
        def _():
            down.wait_recv()
            out_ref[pl.ds(0, 1), :] = (
                0.25 * lhalo_ref[:, :]
                + 0.5 * x_ref[pl.ds(0, 1), :]
                + 0.25 * x_ref[pl.ds(1, 1), :]
            )
            up.wait_send()

        @pl.when(jnp.logical_not(has_right))
        def _():
            out_ref[pl.ds(m - 1, 1), :] = x_ref[pl.ds(m - 1, 1), :]

        @pl.when(has_right)
        def _():
            up.wait_recv()
            out_ref[pl.ds(m - 1, 1), :] = (
                0.25 * x_ref[pl.ds(m - 2, 1), :]
                + 0.5 * x_ref[pl.ds(m - 1, 1), :]
                + 0.25 * rhalo_ref[:, :]
            )
            down.wait_send()

    return pl.pallas_call(
        body,
        out_shape=jax.ShapeDtypeStruct((m, n), x.dtype),
        in_specs=[pl.BlockSpec(memory_space=pltpu.VMEM)],
        out_specs=pl.BlockSpec(memory_space=pltpu.VMEM),
        scratch_shapes=[
            pltpu.VMEM((1, n), x.dtype),
            pltpu.VMEM((1, n), x.dtype),
            pltpu.SemaphoreType.DMA((2,)),
            pltpu.SemaphoreType.DMA((2,)),
        ],
        compiler_params=pltpu.CompilerParams(collective_id=0),
    )(x)
